# baseline (speedup 1.0000x reference)
import functools

import jax
import jax.numpy as jnp
from jax import lax
from jax.experimental import pallas as pl
from jax.experimental.pallas import tpu as pltpu
from jax.experimental.pallas import tpu_sc as plsc

_N = 10000
_E = 320000
_D = 128
_G = 64

_Q = 32
_TW = 32
_TWD = 16
_NT = 16
_NW = 32
_EPW = _E // _NW
_EPT = _E // _NT
_K = 80
_NCH = _EPT // _K
_IRW = _EPW // _K
_IRT = _E // _K
_RB = 1256
_NPAD = 8 * _RB
_RACC = 2 * _NPAD
_RPT = _RACC // _NT
_NZ = 8


def _project(h, w, av):
    xw = jnp.dot(h, w, preferred_element_type=jnp.float32)
    s2 = jnp.dot(xw, av, preferred_element_type=jnp.float32)
    return xw, s2[:, 0:1], s2[:, 1:2]


def _p1_body(h_ref, w_ref, av_ref, xw_ref, ssrc_ref, sdst_ref, s2d_ref):
    xw, ssrc, sdst = _project(h_ref[...], w_ref[...], av_ref[...])
    xw_ref[...] = xw
    ssrc_ref[...] = ssrc
    sdst_ref[...] = sdst
    s2d_ref[0] = ssrc.reshape(_NPAD // 16, 16)
    s2d_ref[1] = sdst.reshape(_NPAD // 16, 16)


_p1 = pl.pallas_call(
    _p1_body,
    out_shape=[
        jax.ShapeDtypeStruct((_NPAD, _D), jnp.float32),
        jax.ShapeDtypeStruct((_NPAD, 1), jnp.float32),
        jax.ShapeDtypeStruct((_NPAD, 1), jnp.float32),
        jax.ShapeDtypeStruct((2, _NPAD // 16, 16), jnp.float32),
    ],
)


def _p2_body(xw_ref, sall_ref, sblk_ref, t0_ref, t1_ref, t2_ref, t3_ref,
             td_ref):
    b = pl.program_id(0)
    cs = jnp.max(sall_ref[...])
    sb = sblk_ref[...]
    dev = jnp.where(b < 8, sb - cs, 0.2 * (sb - cs))
    f = jnp.exp(dev)
    xw = xw_ref[...]
    for qi, t_ref in enumerate((t0_ref, t1_ref, t2_ref, t3_ref)):
        xwq = xw[:, qi * _Q:(qi + 1) * _Q]
        t_ref[...] = f * xwq
    z = jnp.zeros((_RB, _TWD - 1), jnp.float32)
    td_ref[...] = jnp.concatenate([f, z], axis=1)


_p2 = pl.pallas_call(
    _p2_body,
    grid=(16,),
    in_specs=[
        pl.BlockSpec((_RB, _D), lambda b: (b % 8, 0)),
        pl.BlockSpec((_NPAD, 1), lambda b: (0, 0)),
        pl.BlockSpec((_RB, 1), lambda b: (b % 8, 0)),
    ],
    out_specs=[pl.BlockSpec((_RB, _TW), lambda b: (b, 0))] * 4 + [
        pl.BlockSpec((_RB, _TWD), lambda b: (b, 0))],
    out_shape=[jax.ShapeDtypeStruct((_RACC, _TW), jnp.float32)] * 4 + [
        jax.ShapeDtypeStruct((_RACC, _TWD), jnp.float32)],
)


def _index_body(src_h, dst_h, s2d_h, gi_h, si_h,
                srcv, dstv, ssv, sdv, gib, sib):
    cid = lax.axis_index("c")
    sid = lax.axis_index("s")
    w = cid * _NT + sid
    ebase = w * _EPW
    pltpu.sync_copy(src_h.at[pl.ds(ebase, _EPW)], srcv)
    pltpu.sync_copy(dst_h.at[pl.ds(ebase, _EPW)], dstv)
    pltpu.sync_copy(s2d_h.at[0], ssv)
    pltpu.sync_copy(s2d_h.at[1], sdv)

    def idx_body(r, carry):
        for j in range(_K // 16):
            sl = pl.ds(r * _K + j * 16, 16)
            s = srcv[sl]
            d = dstv[sl]
            a = (plsc.load_gather(ssv, [s >> 4, s & 15])
                 + plsc.load_gather(sdv, [d >> 4, d & 15]))
            off = jnp.where(a <= 0.0, _NPAD, 0).astype(jnp.int32)
            gib[r, 0, pl.ds(j * 16, 16)] = s + off
            sib[r, 0, pl.ds(j * 16, 16)] = d + off
        return carry

    lax.fori_loop(0, _IRW, idx_body, 0)
    rbase = w * _IRW
    pltpu.sync_copy(gib, gi_h.at[pl.ds(rbase, _IRW)])
    pltpu.sync_copy(sib, si_h.at[pl.ds(rbase, _IRW)])


_index = functools.partial(
    pl.kernel,
    mesh=plsc.VectorSubcoreMesh(core_axis_name="c", subcore_axis_name="s"),
    compiler_params=pltpu.CompilerParams(needs_layout_passes=False,
                                         use_tc_tiling_on_sc=False),
    out_type=[jax.ShapeDtypeStruct((_IRT, 1, _K), jnp.int32)] * 2,
    scratch_types=[
        pltpu.VMEM((_EPW,), jnp.int32),
        pltpu.VMEM((_EPW,), jnp.int32),
        pltpu.VMEM((_NPAD // 16, 16), jnp.float32),
        pltpu.VMEM((_NPAD // 16, 16), jnp.float32),
        pltpu.VMEM((_IRW, 1, _K), jnp.int32),
        pltpu.VMEM((_IRW, 1, _K), jnp.int32),
    ],
)(_index_body)


def _stream_body(gi_h, si_h, t0_h, t1_h, t2_h, t3_h, td_h,
                 o0_h, o1_h, o2_h, o3_h, od_h,
                 gidx, sidx, gb0, gb1, gb2, gb3, gb4, gd0, gd1, gd2, gd3, gd4,
                 zb, zbd, accf, accd,
                 sg0, sg1, sg2, sg3, sg4, ssem):
    gbs = (gb0, gb1, gb2, gb3, gb4)
    gds = (gd0, gd1, gd2, gd3, gd4)
    sgs = (sg0, sg1, sg2, sg3, sg4)
    cid = lax.axis_index("c")
    sid = lax.axis_index("s")
    ibase = sid * _NCH
    pltpu.sync_copy(gi_h.at[pl.ds(ibase, _NCH)], gidx)
    pltpu.sync_copy(si_h.at[pl.ds(ibase, _NCH)], sidx)

    z16 = jnp.zeros((16,), jnp.float32)

    def zb_body(r, carry):
        for j in range(_TW // 16):
            zb[r, pl.ds(j * 16, 16)] = z16
        zbd[r, :] = z16
        return carry

    lax.fori_loop(0, _NZ, zb_body, 0)

    rbase = sid * _RPT

    def do_pass(t_h, o_h, acc, bufs, sems, nbuf, zbuf):
        def zc_body(k, carry):
            pltpu.sync_copy(zbuf, acc.at[pl.ds(rbase + k * _NZ, _NZ)])
            return carry

        lax.fori_loop(0, _RPT // _NZ, zc_body, 0)
        plsc.subcore_barrier()
        for b in range(nbuf):
            pltpu.async_copy(t_h.at[gidx.at[b, 0]], bufs[b], sems[b])

        def ch_body(t, carry):
            for b in range(nbuf):
                c2 = t * nbuf + b
                pltpu.make_async_copy(t_h.at[gidx.at[c2, 0]],
                                      bufs[b], sems[b]).wait()
                hs = pltpu.async_copy(bufs[b], acc.at[sidx.at[c2, 0]],
                                      ssem, add=True)
                hs.wait()

                @pl.when(c2 + nbuf < _NCH)
                def _():
                    pltpu.async_copy(t_h.at[gidx.at[c2 + nbuf, 0]],
                                     bufs[b], sems[b])
            return carry

        lax.fori_loop(0, _NCH // nbuf, ch_body, 0)
        plsc.subcore_barrier()
        pltpu.sync_copy(acc.at[pl.ds(rbase, _RPT)],
                        o_h.at[pl.ds(rbase, _RPT)])
        plsc.subcore_barrier()

    @pl.when(cid == 0)
    def _():
        do_pass(t0_h, o0_h, accf, gbs, sgs, 5, zb)
        do_pass(t1_h, o1_h, accf, gbs, sgs, 5, zb)
        do_pass(td_h, od_h, accd, gds, sgs, 5, zbd)

    @pl.when(cid == 1)
    def _():
        do_pass(t2_h, o2_h, accf, gbs, sgs, 5, zb)
        do_pass(t3_h, o3_h, accf, gbs, sgs, 5, zb)


_stream = functools.partial(
    pl.kernel,
    mesh=plsc.VectorSubcoreMesh(core_axis_name="c", subcore_axis_name="s"),
    compiler_params=pltpu.CompilerParams(needs_layout_passes=False,
                                         use_tc_tiling_on_sc=False),
    out_type=[jax.ShapeDtypeStruct((_RACC, _TW), jnp.float32)] * 4 + [
        jax.ShapeDtypeStruct((_RACC, _TWD), jnp.float32)],
    scratch_types=[
        pltpu.VMEM((_NCH, 1, _K), jnp.int32),
        pltpu.VMEM((_NCH, 1, _K), jnp.int32),
        pltpu.VMEM((_K, _TW), jnp.float32),
        pltpu.VMEM((_K, _TW), jnp.float32),
        pltpu.VMEM((_K, _TW), jnp.float32),
        pltpu.VMEM((_K, _TW), jnp.float32),
        pltpu.VMEM((_K, _TW), jnp.float32),
        pltpu.VMEM((_K, _TWD), jnp.float32),
        pltpu.VMEM((_K, _TWD), jnp.float32),
        pltpu.VMEM((_K, _TWD), jnp.float32),
        pltpu.VMEM((_K, _TWD), jnp.float32),
        pltpu.VMEM((_K, _TWD), jnp.float32),
        pltpu.VMEM((_NZ, _TW), jnp.float32),
        pltpu.VMEM((_NZ, _TWD), jnp.float32),
        pltpu.VMEM_SHARED((_RACC, _TW), jnp.float32),
        pltpu.VMEM_SHARED((_RACC, _TWD), jnp.float32),
        pltpu.SemaphoreType.DMA,
        pltpu.SemaphoreType.DMA,
        pltpu.SemaphoreType.DMA,
        pltpu.SemaphoreType.DMA,
        pltpu.SemaphoreType.DMA,
        pltpu.SemaphoreType.DMA,
    ],
)(_stream_body)


def _e1_body(o0p_ref, o0m_ref, o1p_ref, o1m_ref, o2p_ref, o2m_ref,
             o3p_ref, o3m_ref, odp_ref, odm_ref, ssrc_ref, sdst_ref,
             sdb_ref, b_ref, gat_ref, ps_ref, pq_ref):
    i = pl.program_id(0)
    cs = jnp.max(ssrc_ref[...])
    cd = jnp.max(sdst_ref[...])
    c = cs + cd
    sd = sdb_ref[...]
    fp = jnp.exp(sd - cd)
    fm = jnp.exp(0.2 * (sd - cd) - 0.8 * c)
    combs = []
    for op_ref, om_ref in ((o0p_ref, o0m_ref), (o1p_ref, o1m_ref),
                           (o2p_ref, o2m_ref), (o3p_ref, o3m_ref)):
        combs.append(fp * op_ref[...] + fm * om_ref[...])
    numer = jnp.concatenate(combs, axis=1)
    den = (fp * odp_ref[...] + fm * odm_ref[...])[:, 0:1]
    gat = numer / (den + 1e-16) + b_ref[...]
    row = lax.broadcasted_iota(jnp.int32, (_RB, _D), 0) + i * _RB
    gat = jnp.where(row < _N, gat, 0.0)
    gat_ref[...] = gat
    ps_ref[...] = jnp.sum(gat, axis=0, keepdims=True).reshape(1, 1, _D)
    pq_ref[...] = jnp.sum(gat * gat, axis=0, keepdims=True).reshape(1, 1, _D)


_e1 = pl.pallas_call(
    _e1_body,
    grid=(8,),
    in_specs=[
        sp for _q in range(4) for sp in (
            pl.BlockSpec((_RB, _TW), lambda i: (i, 0)),
            pl.BlockSpec((_RB, _TW), lambda i: (i + 8, 0)),
        )
    ] + [
        pl.BlockSpec((_RB, _TWD), lambda i: (i, 0)),
        pl.BlockSpec((_RB, _TWD), lambda i: (i + 8, 0)),
    ] + [
        pl.BlockSpec((_NPAD, 1), lambda i: (0, 0)),
        pl.BlockSpec((_NPAD, 1), lambda i: (0, 0)),
        pl.BlockSpec((_RB, 1), lambda i: (i, 0)),
        pl.BlockSpec((1, _D), lambda i: (0, 0)),
    ],
    out_specs=[
        pl.BlockSpec((_RB, _D), lambda i: (i, 0)),
        pl.BlockSpec((1, 1, _D), lambda i: (i, 0, 0)),
        pl.BlockSpec((1, 1, _D), lambda i: (i, 0, 0)),
    ],
    out_shape=[
        jax.ShapeDtypeStruct((_NPAD, _D), jnp.float32),
        jax.ShapeDtypeStruct((8, 1, _D), jnp.float32),
        jax.ShapeDtypeStruct((8, 1, _D), jnp.float32),
    ],
)


def _bn_res(gat_ref, ps_ref, pq_ref, g_ref, be_ref, hprev_ref):
    gat = gat_ref[...]
    m = jnp.sum(ps_ref[...], axis=0) / _N
    m2 = jnp.sum(pq_ref[...], axis=0) / _N
    v = m2 - m * m
    bn = (gat - m) / jnp.sqrt(v + 1e-5) * g_ref[...] + be_ref[...]
    return jnp.maximum(bn, 0.0) + hprev_ref[...]


def _e2p1_body(gat_ref, ps_ref, pq_ref, g_ref, be_ref, hprev_ref,
               w_ref, av_ref, h_ref, xw_ref, ssrc_ref, sdst_ref, s2d_ref):
    h = _bn_res(gat_ref, ps_ref, pq_ref, g_ref, be_ref, hprev_ref)
    h_ref[...] = h
    xw, ssrc, sdst = _project(h, w_ref[...], av_ref[...])
    xw_ref[...] = xw
    ssrc_ref[...] = ssrc
    sdst_ref[...] = sdst
    s2d_ref[0] = ssrc.reshape(_NPAD // 16, 16)
    s2d_ref[1] = sdst.reshape(_NPAD // 16, 16)


_e2p1 = pl.pallas_call(
    _e2p1_body,
    out_shape=[
        jax.ShapeDtypeStruct((_NPAD, _D), jnp.float32),
        jax.ShapeDtypeStruct((_NPAD, _D), jnp.float32),
        jax.ShapeDtypeStruct((_NPAD, 1), jnp.float32),
        jax.ShapeDtypeStruct((_NPAD, 1), jnp.float32),
        jax.ShapeDtypeStruct((2, _NPAD // 16, 16), jnp.float32),
    ],
)


_RBW = _NPAD // _G


def _e2head_body(gat_ref, ps_ref, pq_ref, g_ref, be_ref, hprev_ref,
                 batch_ref, wq_ref, bq_ref, wr_ref, br_ref, out_ref,
                 hmax_ref, band_ref, hbuf_ref):
    h = _bn_res(gat_ref, ps_ref, pq_ref, g_ref, be_ref, hprev_ref)
    hbuf_ref[...] = h
    batch = batch_ref[...]
    gidx_row = lax.broadcasted_iota(jnp.int32, (_NPAD, _G), 1)
    oh = (batch == gidx_row).astype(jnp.float32)
    dn = (((0,), (0,)), ((), ()))
    sums = lax.dot_general(oh, h, dn, preferred_element_type=jnp.float32)
    cnt = lax.dot_general(oh, jnp.ones((_NPAD, 1), jnp.float32), dn,
                          preferred_element_type=jnp.float32)
    mean = sums / jnp.maximum(cnt, 1.0)

    lt = (batch < gidx_row).astype(jnp.float32)
    starts = lax.dot_general(jnp.ones((_NPAD, 1), jnp.float32), lt, dn,
                             preferred_element_type=jnp.float32)
    counts = cnt[:, 0].reshape(1, _G)
    giota = lax.broadcasted_iota(jnp.int32, (1, _G), 1)
    neg_inf = jnp.float32(-jnp.inf)

    def band_body(k, carry):
        band_ref[pl.ds(k, 1), :] = jnp.max(
            hbuf_ref[pl.ds(k * _RBW, _RBW), :], axis=0, keepdims=True)
        return carry

    lax.fori_loop(0, _G, band_body, 0)

    biota = lax.broadcasted_iota(jnp.int32, (_G, _D), 0)
    wiota = lax.broadcasted_iota(jnp.int32, (_RBW, _D), 0)

    def mx_body(gi, carry):
        sel = (giota == gi).astype(jnp.float32)
        st = jnp.sum(starts * sel).astype(jnp.int32)
        cn = jnp.sum(counts * sel).astype(jnp.int32)
        en = st + cn
        js = (st + _RBW - 1) // _RBW
        je = en // _RBW
        inner = jnp.max(jnp.where((biota >= js) & (biota < je),
                                  band_ref[...], neg_inf), axis=0)
        c1 = jnp.minimum(st, _NPAD - _RBW)
        w1 = hbuf_ref[pl.ds(c1, _RBW), :]
        r1 = wiota + c1
        m1 = jnp.max(jnp.where((r1 >= st) & (r1 < en), w1, neg_inf), axis=0)
        c2 = jnp.clip(en - _RBW, 0, _NPAD - _RBW)
        w2 = hbuf_ref[pl.ds(c2, _RBW), :]
        r2 = wiota + c2
        m2 = jnp.max(jnp.where((r2 >= st) & (r2 < en), w2, neg_inf), axis=0)
        hm = jnp.maximum(jnp.maximum(inner, m1), m2)
        hmax_ref[pl.ds(gi, 1), :] = hm.reshape(1, _D)
        return carry

    lax.fori_loop(0, _G, mx_body, 0)
    hmax = hmax_ref[...]
    hmax = jnp.where(jnp.isfinite(hmax), hmax, 0.0)
    pooled = jnp.concatenate([mean, hmax], axis=1)
    hq = jnp.tanh(jnp.dot(pooled, wq_ref[...],
                          preferred_element_type=jnp.float32) + bq_ref[...])
    comb = jnp.concatenate([pooled, hq], axis=1)
    out_ref[...] = jnp.dot(comb, wr_ref[...],
                           preferred_element_type=jnp.float32) + br_ref[...]


_e2head = pl.pallas_call(
    _e2head_body,
    out_shape=jax.ShapeDtypeStruct((_G, 10), jnp.float32),
    scratch_shapes=[pltpu.VMEM((_G, _D), jnp.float32),
                    pltpu.VMEM((_G, _D), jnp.float32),
                    pltpu.VMEM((_NPAD, _D), jnp.float32)],
)


def kernel(x, edge_index, batch, W1, asrc1, adst1, b1, g1, be1,
           W2, asrc2, adst2, b2, g2, be2, Wq, bq, Wr, br):
    src = edge_index[0]
    dst = edge_index[1]
    xp = jnp.pad(x, ((0, _NPAD - _N), (0, 0)))
    batch_p = jnp.pad(batch, (0, _NPAD - _N),
                      constant_values=_G).reshape(_NPAD, 1)
    av1 = jnp.stack([asrc1, adst1], axis=1)
    av2 = jnp.stack([asrc2, adst2], axis=1)

    xw, ssrc, sdst, s2d = _p1(xp, W1, av1)
    t0, t1, t2, t3, td = _p2(xw, ssrc, ssrc)
    gi, si = _index(src, dst, s2d)
    o0, o1, o2, o3, od = _stream(gi, si, t0, t1, t2, t3, td)
    gat, ps, pq = _e1(o0, o0, o1, o1, o2, o2, o3, o3, od, od,
                      ssrc, sdst, sdst, b1.reshape(1, _D))
    h1, xw, ssrc, sdst, s2d = _e2p1(gat, ps, pq, g1.reshape(1, _D),
                                    be1.reshape(1, _D), xp, W2, av2)
    t0, t1, t2, t3, td = _p2(xw, ssrc, ssrc)
    gi, si = _index(src, dst, s2d)
    o0, o1, o2, o3, od = _stream(gi, si, t0, t1, t2, t3, td)
    gat, ps, pq = _e1(o0, o0, o1, o1, o2, o2, o3, o3, od, od,
                      ssrc, sdst, sdst, b2.reshape(1, _D))
    return _e2head(gat, ps, pq, g2.reshape(1, _D), be2.reshape(1, _D), h1,
                   batch_p, Wq, bq.reshape(1, -1), Wr, br.reshape(1, -1))


# speedup vs baseline: 1.0858x; 1.0858x over previous
import functools

import jax
import jax.numpy as jnp
from jax import lax
from jax.experimental import pallas as pl
from jax.experimental.pallas import tpu as pltpu
from jax.experimental.pallas import tpu_sc as plsc

_N = 10000
_E = 320000
_D = 128
_G = 64

_Q = 64
_TW = 80
_IW = 10
_NT = 16
_NW = 32
_EPW = _E // _NW
_EPT = _E // _NT
_K = 80
_NCH = _EPT // _K
_IRW = _EPW // _K
_IRT = _E // _K
_RB = 1256
_NPAD = 8 * _RB
_RACC = 2 * _NPAD
_RPT = _RACC // _NT
_NZ = 8


def _project(h, w, av):
    xw = jnp.dot(h, w, preferred_element_type=jnp.float32)
    s2 = jnp.dot(xw, av, preferred_element_type=jnp.float32)
    return xw, s2[:, 0:1], s2[:, 1:2]


def _p1_body(h_ref, w_ref, av_ref, xw_ref, ssrc_ref, sdst_ref, s2d_ref):
    xw, ssrc, sdst = _project(h_ref[...], w_ref[...], av_ref[...])
    xw_ref[...] = xw
    ssrc_ref[...] = ssrc
    sdst_ref[...] = sdst
    s2d_ref[0] = ssrc.reshape(_NPAD // 16, 16)
    s2d_ref[1] = sdst.reshape(_NPAD // 16, 16)


_p1 = pl.pallas_call(
    _p1_body,
    out_shape=[
        jax.ShapeDtypeStruct((_NPAD, _D), jnp.float32),
        jax.ShapeDtypeStruct((_NPAD, 1), jnp.float32),
        jax.ShapeDtypeStruct((_NPAD, 1), jnp.float32),
        jax.ShapeDtypeStruct((2, _NPAD // 16, 16), jnp.float32),
    ],
)


def _p2_body(xw_ref, sall_ref, sblk_ref, t0_ref, t1_ref):
    b = pl.program_id(0)
    cs = jnp.max(sall_ref[...])
    sb = sblk_ref[...]
    dev = jnp.where(b < 8, sb - cs, 0.2 * (sb - cs))
    f = jnp.exp(dev)
    xw = xw_ref[...]
    z = jnp.zeros((_RB, _TW - _Q - 1), jnp.float32)
    for qi, t_ref in enumerate((t0_ref, t1_ref)):
        xwq = xw[:, qi * _Q:(qi + 1) * _Q]
        t_ref[...] = jnp.concatenate([f * xwq, f, z], axis=1)


_p2 = pl.pallas_call(
    _p2_body,
    grid=(16,),
    in_specs=[
        pl.BlockSpec((_RB, _D), lambda b: (b % 8, 0)),
        pl.BlockSpec((_NPAD, 1), lambda b: (0, 0)),
        pl.BlockSpec((_RB, 1), lambda b: (b % 8, 0)),
    ],
    out_specs=[pl.BlockSpec((_RB, _TW), lambda b: (b, 0))] * 2,
    out_shape=[jax.ShapeDtypeStruct((_RACC, _TW), jnp.float32)] * 2,
)


def _index_body(src_h, dst_h, s2d_h, gi_h, si_h,
                srcv, dstv, ssv, sdv, gib, sib):
    cid = lax.axis_index("c")
    sid = lax.axis_index("s")
    w = cid * _NT + sid
    ebase = w * _EPW
    pltpu.sync_copy(src_h.at[pl.ds(ebase, _EPW)], srcv)
    pltpu.sync_copy(dst_h.at[pl.ds(ebase, _EPW)], dstv)
    pltpu.sync_copy(s2d_h.at[0], ssv)
    pltpu.sync_copy(s2d_h.at[1], sdv)

    def idx_body(r, carry):
        for j in range(_K // 16):
            sl = pl.ds(r * _K + j * 16, 16)
            s = srcv[sl]
            d = dstv[sl]
            a = (plsc.load_gather(ssv, [s >> 4, s & 15])
                 + plsc.load_gather(sdv, [d >> 4, d & 15]))
            off = jnp.where(a <= 0.0, _NPAD, 0).astype(jnp.int32)
            gib[r, 0, pl.ds(j * 16, 16)] = s + off
            sib[r, 0, pl.ds(j * 16, 16)] = d + off
        return carry

    lax.fori_loop(0, _IRW, idx_body, 0)
    rbase = w * _IRW
    pltpu.sync_copy(gib, gi_h.at[pl.ds(rbase, _IRW)])
    pltpu.sync_copy(sib, si_h.at[pl.ds(rbase, _IRW)])


_index = functools.partial(
    pl.kernel,
    mesh=plsc.VectorSubcoreMesh(core_axis_name="c", subcore_axis_name="s"),
    compiler_params=pltpu.CompilerParams(needs_layout_passes=False,
                                         use_tc_tiling_on_sc=False),
    out_type=[jax.ShapeDtypeStruct((_IRT, 1, _K), jnp.int32)] * 2,
    scratch_types=[
        pltpu.VMEM((_EPW,), jnp.int32),
        pltpu.VMEM((_EPW,), jnp.int32),
        pltpu.VMEM((_NPAD // 16, 16), jnp.float32),
        pltpu.VMEM((_NPAD // 16, 16), jnp.float32),
        pltpu.VMEM((_IRW, 1, _K), jnp.int32),
        pltpu.VMEM((_IRW, 1, _K), jnp.int32),
    ],
)(_index_body)


def _stream_body(gi_h, si_h, t0_h, t1_h, o0_h, o1_h,
                 gidx, sidx, gb0, gb1, zb, acc, sg0, sg1, ssem):
    gbs = (gb0, gb1)
    sgs = (sg0, sg1)
    cid = lax.axis_index("c")
    sid = lax.axis_index("s")
    ibase = sid * _NCH

    z16 = jnp.zeros((16,), jnp.float32)

    def zb_body(r, carry):
        for j in range(_TW // 16):
            zb[r, pl.ds(j * 16, 16)] = z16
        return carry

    lax.fori_loop(0, _NZ, zb_body, 0)

    rbase = sid * _RPT

    def zc_body(k, carry):
        pltpu.sync_copy(zb, acc.at[pl.ds(rbase + k * _NZ, _NZ)])
        return carry

    lax.fori_loop(0, _RPT // _NZ, zc_body, 0)
    plsc.subcore_barrier()

    def run(t_h, o_h):
        pltpu.sync_copy(gi_h.at[pl.ds(ibase, _IW)], gidx.at[pl.ds(0, _IW)])
        pltpu.sync_copy(si_h.at[pl.ds(ibase, _IW)], sidx.at[pl.ds(0, _IW)])
        for b in range(2):
            pltpu.async_copy(t_h.at[gidx.at[b, 0]], gbs[b], sgs[b])

        def win_body(wdw, carry):
            wslot = wdw % 2
            nslot = (wdw + 1) % 2

            @pl.when(wdw + 1 < _NCH // _IW)
            def _():
                pltpu.sync_copy(
                    gi_h.at[pl.ds(ibase + (wdw + 1) * _IW, _IW)],
                    gidx.at[pl.ds(nslot * _IW, _IW)])
                pltpu.sync_copy(
                    si_h.at[pl.ds(ibase + (wdw + 1) * _IW, _IW)],
                    sidx.at[pl.ds(nslot * _IW, _IW)])

            def ch_body(t, carry2):
                for b in range(2):
                    k2 = t * 2 + b
                    c2 = wdw * _IW + k2
                    lrow = wslot * _IW + k2
                    pltpu.make_async_copy(t_h.at[gidx.at[lrow, 0]],
                                          gbs[b], sgs[b]).wait()
                    hs = pltpu.async_copy(gbs[b], acc.at[sidx.at[lrow, 0]],
                                          ssem, add=True)
                    hs.wait()

                    @pl.when(c2 + 2 < _NCH)
                    def _():
                        nk = k2 + 2
                        nrow = jnp.where(nk < _IW,
                                         wslot * _IW + nk,
                                         nslot * _IW + nk - _IW)
                        pltpu.async_copy(t_h.at[gidx.at[nrow, 0]],
                                         gbs[b], sgs[b])
                return carry2

            lax.fori_loop(0, _IW // 2, ch_body, 0)
            return carry

        lax.fori_loop(0, _NCH // _IW, win_body, 0)
        plsc.subcore_barrier()
        pltpu.sync_copy(acc.at[pl.ds(rbase, _RPT)],
                        o_h.at[pl.ds(rbase, _RPT)])

    @pl.when(cid == 0)
    def _():
        run(t0_h, o0_h)

    @pl.when(cid == 1)
    def _():
        run(t1_h, o1_h)


_stream = functools.partial(
    pl.kernel,
    mesh=plsc.VectorSubcoreMesh(core_axis_name="c", subcore_axis_name="s"),
    compiler_params=pltpu.CompilerParams(needs_layout_passes=False,
                                         use_tc_tiling_on_sc=False),
    out_type=[jax.ShapeDtypeStruct((_RACC, _TW), jnp.float32)] * 2,
    scratch_types=[
        pltpu.VMEM((2 * _IW, 1, _K), jnp.int32),
        pltpu.VMEM((2 * _IW, 1, _K), jnp.int32),
        pltpu.VMEM((_K, _TW), jnp.float32),
        pltpu.VMEM((_K, _TW), jnp.float32),
        pltpu.VMEM((_NZ, _TW), jnp.float32),
        pltpu.VMEM_SHARED((_RACC, _TW), jnp.float32),
        pltpu.SemaphoreType.DMA,
        pltpu.SemaphoreType.DMA,
        pltpu.SemaphoreType.DMA,
    ],
)(_stream_body)


def _e1_body(o0p_ref, o0m_ref, o1p_ref, o1m_ref, ssrc_ref, sdst_ref,
             sdb_ref, b_ref, gat_ref, ps_ref, pq_ref):
    i = pl.program_id(0)
    cs = jnp.max(ssrc_ref[...])
    cd = jnp.max(sdst_ref[...])
    c = cs + cd
    sd = sdb_ref[...]
    fp = jnp.exp(sd - cd)
    fm = jnp.exp(0.2 * (sd - cd) - 0.8 * c)
    combs = []
    for op_ref, om_ref in ((o0p_ref, o0m_ref), (o1p_ref, o1m_ref)):
        combs.append(fp * op_ref[...] + fm * om_ref[...])
    numer = jnp.concatenate([cb[:, :_Q] for cb in combs], axis=1)
    den = combs[0][:, _Q:_Q + 1]
    gat = numer / (den + 1e-16) + b_ref[...]
    row = lax.broadcasted_iota(jnp.int32, (_RB, _D), 0) + i * _RB
    gat = jnp.where(row < _N, gat, 0.0)
    gat_ref[...] = gat
    ps_ref[...] = jnp.sum(gat, axis=0, keepdims=True).reshape(1, 1, _D)
    pq_ref[...] = jnp.sum(gat * gat, axis=0, keepdims=True).reshape(1, 1, _D)


_e1 = pl.pallas_call(
    _e1_body,
    grid=(8,),
    in_specs=[
        sp for _q in range(2) for sp in (
            pl.BlockSpec((_RB, _TW), lambda i: (i, 0)),
            pl.BlockSpec((_RB, _TW), lambda i: (i + 8, 0)),
        )
    ] + [
        pl.BlockSpec((_NPAD, 1), lambda i: (0, 0)),
        pl.BlockSpec((_NPAD, 1), lambda i: (0, 0)),
        pl.BlockSpec((_RB, 1), lambda i: (i, 0)),
        pl.BlockSpec((1, _D), lambda i: (0, 0)),
    ],
    out_specs=[
        pl.BlockSpec((_RB, _D), lambda i: (i, 0)),
        pl.BlockSpec((1, 1, _D), lambda i: (i, 0, 0)),
        pl.BlockSpec((1, 1, _D), lambda i: (i, 0, 0)),
    ],
    out_shape=[
        jax.ShapeDtypeStruct((_NPAD, _D), jnp.float32),
        jax.ShapeDtypeStruct((8, 1, _D), jnp.float32),
        jax.ShapeDtypeStruct((8, 1, _D), jnp.float32),
    ],
)


def _bn_res(gat_ref, ps_ref, pq_ref, g_ref, be_ref, hprev_ref):
    gat = gat_ref[...]
    m = jnp.sum(ps_ref[...], axis=0) / _N
    m2 = jnp.sum(pq_ref[...], axis=0) / _N
    v = m2 - m * m
    bn = (gat - m) / jnp.sqrt(v + 1e-5) * g_ref[...] + be_ref[...]
    return jnp.maximum(bn, 0.0) + hprev_ref[...]


def _e2p1_body(gat_ref, ps_ref, pq_ref, g_ref, be_ref, hprev_ref,
               w_ref, av_ref, h_ref, xw_ref, ssrc_ref, sdst_ref, s2d_ref):
    h = _bn_res(gat_ref, ps_ref, pq_ref, g_ref, be_ref, hprev_ref)
    h_ref[...] = h
    xw, ssrc, sdst = _project(h, w_ref[...], av_ref[...])
    xw_ref[...] = xw
    ssrc_ref[...] = ssrc
    sdst_ref[...] = sdst
    s2d_ref[0] = ssrc.reshape(_NPAD // 16, 16)
    s2d_ref[1] = sdst.reshape(_NPAD // 16, 16)


_e2p1 = pl.pallas_call(
    _e2p1_body,
    out_shape=[
        jax.ShapeDtypeStruct((_NPAD, _D), jnp.float32),
        jax.ShapeDtypeStruct((_NPAD, _D), jnp.float32),
        jax.ShapeDtypeStruct((_NPAD, 1), jnp.float32),
        jax.ShapeDtypeStruct((_NPAD, 1), jnp.float32),
        jax.ShapeDtypeStruct((2, _NPAD // 16, 16), jnp.float32),
    ],
)


_RBW = _NPAD // _G


def _e2head_body(gat_ref, ps_ref, pq_ref, g_ref, be_ref, hprev_ref,
                 batch_ref, wq_ref, bq_ref, wr_ref, br_ref, out_ref,
                 hmax_ref, band_ref, hbuf_ref):
    h = _bn_res(gat_ref, ps_ref, pq_ref, g_ref, be_ref, hprev_ref)
    hbuf_ref[...] = h
    batch = batch_ref[...]
    gidx_row = lax.broadcasted_iota(jnp.int32, (_NPAD, _G), 1)
    oh = (batch == gidx_row).astype(jnp.float32)
    dn = (((0,), (0,)), ((), ()))
    sums = lax.dot_general(oh, h, dn, preferred_element_type=jnp.float32)
    cnt = lax.dot_general(oh, jnp.ones((_NPAD, 1), jnp.float32), dn,
                          preferred_element_type=jnp.float32)
    mean = sums / jnp.maximum(cnt, 1.0)

    lt = (batch < gidx_row).astype(jnp.float32)
    starts = lax.dot_general(jnp.ones((_NPAD, 1), jnp.float32), lt, dn,
                             preferred_element_type=jnp.float32)
    counts = cnt[:, 0].reshape(1, _G)
    giota = lax.broadcasted_iota(jnp.int32, (1, _G), 1)
    neg_inf = jnp.float32(-jnp.inf)

    def band_body(k, carry):
        band_ref[pl.ds(k, 1), :] = jnp.max(
            hbuf_ref[pl.ds(k * _RBW, _RBW), :], axis=0, keepdims=True)
        return carry

    lax.fori_loop(0, _G, band_body, 0)

    biota = lax.broadcasted_iota(jnp.int32, (_G, _D), 0)
    wiota = lax.broadcasted_iota(jnp.int32, (_RBW, _D), 0)

    def mx_body(gi, carry):
        sel = (giota == gi).astype(jnp.float32)
        st = jnp.sum(starts * sel).astype(jnp.int32)
        cn = jnp.sum(counts * sel).astype(jnp.int32)
        en = st + cn
        js = (st + _RBW - 1) // _RBW
        je = en // _RBW
        inner = jnp.max(jnp.where((biota >= js) & (biota < je),
                                  band_ref[...], neg_inf), axis=0)
        c1 = jnp.minimum(st, _NPAD - _RBW)
        w1 = hbuf_ref[pl.ds(c1, _RBW), :]
        r1 = wiota + c1
        m1 = jnp.max(jnp.where((r1 >= st) & (r1 < en), w1, neg_inf), axis=0)
        c2 = jnp.clip(en - _RBW, 0, _NPAD - _RBW)
        w2 = hbuf_ref[pl.ds(c2, _RBW), :]
        r2 = wiota + c2
        m2 = jnp.max(jnp.where((r2 >= st) & (r2 < en), w2, neg_inf), axis=0)
        hm = jnp.maximum(jnp.maximum(inner, m1), m2)
        hmax_ref[pl.ds(gi, 1), :] = hm.reshape(1, _D)
        return carry

    lax.fori_loop(0, _G, mx_body, 0)
    hmax = hmax_ref[...]
    hmax = jnp.where(jnp.isfinite(hmax), hmax, 0.0)
    pooled = jnp.concatenate([mean, hmax], axis=1)
    hq = jnp.tanh(jnp.dot(pooled, wq_ref[...],
                          preferred_element_type=jnp.float32) + bq_ref[...])
    comb = jnp.concatenate([pooled, hq], axis=1)
    out_ref[...] = jnp.dot(comb, wr_ref[...],
                           preferred_element_type=jnp.float32) + br_ref[...]


_e2head = pl.pallas_call(
    _e2head_body,
    out_shape=jax.ShapeDtypeStruct((_G, 10), jnp.float32),
    scratch_shapes=[pltpu.VMEM((_G, _D), jnp.float32),
                    pltpu.VMEM((_G, _D), jnp.float32),
                    pltpu.VMEM((_NPAD, _D), jnp.float32)],
)


def kernel(x, edge_index, batch, W1, asrc1, adst1, b1, g1, be1,
           W2, asrc2, adst2, b2, g2, be2, Wq, bq, Wr, br):
    src = edge_index[0]
    dst = edge_index[1]
    xp = jnp.pad(x, ((0, _NPAD - _N), (0, 0)))
    batch_p = jnp.pad(batch, (0, _NPAD - _N),
                      constant_values=_G).reshape(_NPAD, 1)
    av1 = jnp.stack([asrc1, adst1], axis=1)
    av2 = jnp.stack([asrc2, adst2], axis=1)

    xw, ssrc, sdst, s2d = _p1(xp, W1, av1)
    t0, t1 = _p2(xw, ssrc, ssrc)
    gi, si = _index(src, dst, s2d)
    o0, o1 = _stream(gi, si, t0, t1)
    gat, ps, pq = _e1(o0, o0, o1, o1,
                      ssrc, sdst, sdst, b1.reshape(1, _D))
    h1, xw, ssrc, sdst, s2d = _e2p1(gat, ps, pq, g1.reshape(1, _D),
                                    be1.reshape(1, _D), xp, W2, av2)
    t0, t1 = _p2(xw, ssrc, ssrc)
    gi, si = _index(src, dst, s2d)
    o0, o1 = _stream(gi, si, t0, t1)
    gat, ps, pq = _e1(o0, o0, o1, o1,
                      ssrc, sdst, sdst, b2.reshape(1, _D))
    return _e2head(gat, ps, pq, g2.reshape(1, _D), be2.reshape(1, _D), h1,
                   batch_p, Wq, bq.reshape(1, -1), Wr, br.reshape(1, -1))


# speedup vs baseline: 1.2748x; 1.1740x over previous
import functools

import jax
import jax.numpy as jnp
from jax import lax
from jax.experimental import pallas as pl
from jax.experimental.pallas import tpu as pltpu
from jax.experimental.pallas import tpu_sc as plsc

_N = 10000
_E = 320000
_D = 128
_G = 64

_Q = 64
_TW = 80
_IW = 10
_NT = 16
_NW = 32
_EPW = _E // _NW
_EPT = _E // _NT
_K = 80
_NCH = _EPT // _K
_IRW = _EPW // _K
_IRT = _E // _K
_RB = 1256
_NPAD = 8 * _RB
_RACC = 2 * _NPAD
_RPT = _RACC // _NT
_NZ = 8


def _project(h, w, av):
    xw = jnp.dot(h, w, preferred_element_type=jnp.float32)
    s2 = jnp.dot(xw, av, preferred_element_type=jnp.float32)
    return xw, s2[:, 0:1], s2[:, 1:2]


def _p1_body(h_ref, w_ref, av_ref, xw_ref, ssrc_ref, sdst_ref, s2d_ref):
    xw, ssrc, sdst = _project(h_ref[...], w_ref[...], av_ref[...])
    xw_ref[...] = xw
    ssrc_ref[...] = ssrc
    sdst_ref[...] = sdst
    s2d_ref[0] = ssrc.reshape(_NPAD // 16, 16)
    s2d_ref[1] = sdst.reshape(_NPAD // 16, 16)


_p1 = pl.pallas_call(
    _p1_body,
    out_shape=[
        jax.ShapeDtypeStruct((_NPAD, _D), jnp.float32),
        jax.ShapeDtypeStruct((_NPAD, 1), jnp.float32),
        jax.ShapeDtypeStruct((_NPAD, 1), jnp.float32),
        jax.ShapeDtypeStruct((2, _NPAD // 16, 16), jnp.float32),
    ],
)


def _p2_body(xw_ref, sall_ref, sblk_ref, t0_ref, t1_ref):
    b = pl.program_id(0)
    cs = jnp.max(sall_ref[...])
    sb = sblk_ref[...]
    dev = jnp.where(b < 8, sb - cs, 0.2 * (sb - cs))
    f = jnp.exp(dev)
    xw = xw_ref[...]
    z = jnp.zeros((_RB, _TW - _Q - 1), jnp.float32)
    for qi, t_ref in enumerate((t0_ref, t1_ref)):
        xwq = xw[:, qi * _Q:(qi + 1) * _Q]
        t_ref[...] = jnp.concatenate([f * xwq, f, z], axis=1)


_p2 = pl.pallas_call(
    _p2_body,
    grid=(16,),
    in_specs=[
        pl.BlockSpec((_RB, _D), lambda b: (b % 8, 0)),
        pl.BlockSpec((_NPAD, 1), lambda b: (0, 0)),
        pl.BlockSpec((_RB, 1), lambda b: (b % 8, 0)),
    ],
    out_specs=[pl.BlockSpec((_RB, _TW), lambda b: (b, 0))] * 2,
    out_shape=[jax.ShapeDtypeStruct((_RACC, _TW), jnp.float32)] * 2,
)


def _index_body(src_h, dst_h, s2d_h, gi_h, si_h,
                srcv, dstv, ssv, sdv, gib, sib):
    cid = lax.axis_index("c")
    sid = lax.axis_index("s")
    w = cid * _NT + sid
    ebase = w * _EPW
    pltpu.sync_copy(src_h.at[pl.ds(ebase, _EPW)], srcv)
    pltpu.sync_copy(dst_h.at[pl.ds(ebase, _EPW)], dstv)
    pltpu.sync_copy(s2d_h.at[0], ssv)
    pltpu.sync_copy(s2d_h.at[1], sdv)

    def idx_body(r, carry):
        for j in range(_K // 16):
            sl = pl.ds(r * _K + j * 16, 16)
            s = srcv[sl]
            d = dstv[sl]
            a = (plsc.load_gather(ssv, [s >> 4, s & 15])
                 + plsc.load_gather(sdv, [d >> 4, d & 15]))
            off = jnp.where(a <= 0.0, _NPAD, 0).astype(jnp.int32)
            gib[r, 0, pl.ds(j * 16, 16)] = s + off
            sib[r, 0, pl.ds(j * 16, 16)] = d + off
        return carry

    lax.fori_loop(0, _IRW, idx_body, 0)
    rbase = w * _IRW
    pltpu.sync_copy(gib, gi_h.at[pl.ds(rbase, _IRW)])
    pltpu.sync_copy(sib, si_h.at[pl.ds(rbase, _IRW)])


_index = functools.partial(
    pl.kernel,
    mesh=plsc.VectorSubcoreMesh(core_axis_name="c", subcore_axis_name="s"),
    compiler_params=pltpu.CompilerParams(needs_layout_passes=False,
                                         use_tc_tiling_on_sc=False),
    out_type=[jax.ShapeDtypeStruct((_IRT, 1, _K), jnp.int32)] * 2,
    scratch_types=[
        pltpu.VMEM((_EPW,), jnp.int32),
        pltpu.VMEM((_EPW,), jnp.int32),
        pltpu.VMEM((_NPAD // 16, 16), jnp.float32),
        pltpu.VMEM((_NPAD // 16, 16), jnp.float32),
        pltpu.VMEM((_IRW, 1, _K), jnp.int32),
        pltpu.VMEM((_IRW, 1, _K), jnp.int32),
    ],
)(_index_body)


def _stream_body(gi_h, si_h, t0_h, t1_h, o0_h, o1_h,
                 gidx, sidx, gb0, gb1, gb2, gb3, acc, sg0, sg1, sg2, sg3,
                 ssem):
    gbs = (gb0, gb1, gb2, gb3)
    sgs = (sg0, sg1, sg2, sg3)
    cid = lax.axis_index("c")
    sid = lax.axis_index("s")
    ibase = sid * _NCH

    z16 = jnp.zeros((16,), jnp.float32)
    for r in range(_NZ):
        for j in range(_TW // 16):
            gb0[r, pl.ds(j * 16, 16)] = z16

    rbase = sid * _RPT

    def zc_body(k, carry):
        pltpu.sync_copy(gb0.at[pl.ds(0, _NZ)],
                        acc.at[pl.ds(rbase + k * _NZ, _NZ)])
        return carry

    lax.fori_loop(0, _RPT // _NZ, zc_body, 0)
    plsc.subcore_barrier()

    def run(t_h, o_h):
        for wdw in range(2):
            pltpu.sync_copy(gi_h.at[pl.ds(ibase + wdw * _IW, _IW)],
                            gidx.at[pl.ds(wdw * _IW, _IW)])
            pltpu.sync_copy(si_h.at[pl.ds(ibase + wdw * _IW, _IW)],
                            sidx.at[pl.ds(wdw * _IW, _IW)])
        for b in range(4):
            pltpu.async_copy(t_h.at[gidx.at[b, 0]], gbs[b], sgs[b])

        def ch_body(t, carry):
            for b in range(4):
                c2 = t * 4 + b

                @pl.when((lax.rem(c2, _IW) == 0) & (c2 + 2 * _IW <= _NCH))
                def _():
                    wnxt = c2 // _IW + 1
                    slot = lax.rem(wnxt, 2) * _IW
                    pltpu.sync_copy(
                        gi_h.at[pl.ds(ibase + wnxt * _IW, _IW)],
                        gidx.at[pl.ds(slot, _IW)])
                    pltpu.sync_copy(
                        si_h.at[pl.ds(ibase + wnxt * _IW, _IW)],
                        sidx.at[pl.ds(slot, _IW)])

                lrow = lax.rem(c2 // _IW, 2) * _IW + lax.rem(c2, _IW)
                pltpu.make_async_copy(t_h.at[gidx.at[lrow, 0]],
                                      gbs[b], sgs[b]).wait()
                hs = pltpu.async_copy(gbs[b], acc.at[sidx.at[lrow, 0]],
                                      ssem, add=True)
                hs.wait()

                @pl.when(c2 + 4 < _NCH)
                def _():
                    nc = c2 + 4
                    nrow = lax.rem(nc // _IW, 2) * _IW + lax.rem(nc, _IW)
                    pltpu.async_copy(t_h.at[gidx.at[nrow, 0]], gbs[b], sgs[b])
            return carry

        lax.fori_loop(0, (_NCH - 2) // 4, ch_body, 0)
        for b, c2 in ((0, _NCH - 2), (1, _NCH - 1)):
            lrow = ((c2 // _IW) % 2) * _IW + c2 % _IW
            pltpu.make_async_copy(t_h.at[gidx.at[lrow, 0]],
                                  gbs[b], sgs[b]).wait()
            pltpu.async_copy(gbs[b], acc.at[sidx.at[lrow, 0]],
                             ssem, add=True).wait()
        plsc.subcore_barrier()
        pltpu.sync_copy(acc.at[pl.ds(rbase, _RPT)],
                        o_h.at[pl.ds(rbase, _RPT)])

    @pl.when(cid == 0)
    def _():
        run(t0_h, o0_h)

    @pl.when(cid == 1)
    def _():
        run(t1_h, o1_h)


_stream = functools.partial(
    pl.kernel,
    mesh=plsc.VectorSubcoreMesh(core_axis_name="c", subcore_axis_name="s"),
    compiler_params=pltpu.CompilerParams(needs_layout_passes=False,
                                         use_tc_tiling_on_sc=False),
    out_type=[jax.ShapeDtypeStruct((_RACC, _TW), jnp.float32)] * 2,
    scratch_types=[
        pltpu.VMEM((2 * _IW, 1, _K), jnp.int32),
        pltpu.VMEM((2 * _IW, 1, _K), jnp.int32),
        pltpu.VMEM((_K, _TW), jnp.float32),
        pltpu.VMEM((_K, _TW), jnp.float32),
        pltpu.VMEM((_K, _TW), jnp.float32),
        pltpu.VMEM((_K, _TW), jnp.float32),
        pltpu.VMEM_SHARED((_RACC, _TW), jnp.float32),
        pltpu.SemaphoreType.DMA,
        pltpu.SemaphoreType.DMA,
        pltpu.SemaphoreType.DMA,
        pltpu.SemaphoreType.DMA,
        pltpu.SemaphoreType.DMA,
    ],
)(_stream_body)


def _e1_body(o0p_ref, o0m_ref, o1p_ref, o1m_ref, ssrc_ref, sdst_ref,
             sdb_ref, b_ref, gat_ref, ps_ref, pq_ref):
    i = pl.program_id(0)
    cs = jnp.max(ssrc_ref[...])
    cd = jnp.max(sdst_ref[...])
    c = cs + cd
    sd = sdb_ref[...]
    fp = jnp.exp(sd - cd)
    fm = jnp.exp(0.2 * (sd - cd) - 0.8 * c)
    combs = []
    for op_ref, om_ref in ((o0p_ref, o0m_ref), (o1p_ref, o1m_ref)):
        combs.append(fp * op_ref[...] + fm * om_ref[...])
    numer = jnp.concatenate([cb[:, :_Q] for cb in combs], axis=1)
    den = combs[0][:, _Q:_Q + 1]
    gat = numer / (den + 1e-16) + b_ref[...]
    row = lax.broadcasted_iota(jnp.int32, (_RB, _D), 0) + i * _RB
    gat = jnp.where(row < _N, gat, 0.0)
    gat_ref[...] = gat
    ps_ref[...] = jnp.sum(gat, axis=0, keepdims=True).reshape(1, 1, _D)
    pq_ref[...] = jnp.sum(gat * gat, axis=0, keepdims=True).reshape(1, 1, _D)


_e1 = pl.pallas_call(
    _e1_body,
    grid=(8,),
    in_specs=[
        sp for _q in range(2) for sp in (
            pl.BlockSpec((_RB, _TW), lambda i: (i, 0)),
            pl.BlockSpec((_RB, _TW), lambda i: (i + 8, 0)),
        )
    ] + [
        pl.BlockSpec((_NPAD, 1), lambda i: (0, 0)),
        pl.BlockSpec((_NPAD, 1), lambda i: (0, 0)),
        pl.BlockSpec((_RB, 1), lambda i: (i, 0)),
        pl.BlockSpec((1, _D), lambda i: (0, 0)),
    ],
    out_specs=[
        pl.BlockSpec((_RB, _D), lambda i: (i, 0)),
        pl.BlockSpec((1, 1, _D), lambda i: (i, 0, 0)),
        pl.BlockSpec((1, 1, _D), lambda i: (i, 0, 0)),
    ],
    out_shape=[
        jax.ShapeDtypeStruct((_NPAD, _D), jnp.float32),
        jax.ShapeDtypeStruct((8, 1, _D), jnp.float32),
        jax.ShapeDtypeStruct((8, 1, _D), jnp.float32),
    ],
)


def _bn_res(gat_ref, ps_ref, pq_ref, g_ref, be_ref, hprev_ref):
    gat = gat_ref[...]
    m = jnp.sum(ps_ref[...], axis=0) / _N
    m2 = jnp.sum(pq_ref[...], axis=0) / _N
    v = m2 - m * m
    bn = (gat - m) / jnp.sqrt(v + 1e-5) * g_ref[...] + be_ref[...]
    return jnp.maximum(bn, 0.0) + hprev_ref[...]


def _e2p1_body(gat_ref, ps_ref, pq_ref, g_ref, be_ref, hprev_ref,
               w_ref, av_ref, h_ref, xw_ref, ssrc_ref, sdst_ref, s2d_ref):
    h = _bn_res(gat_ref, ps_ref, pq_ref, g_ref, be_ref, hprev_ref)
    h_ref[...] = h
    xw, ssrc, sdst = _project(h, w_ref[...], av_ref[...])
    xw_ref[...] = xw
    ssrc_ref[...] = ssrc
    sdst_ref[...] = sdst
    s2d_ref[0] = ssrc.reshape(_NPAD // 16, 16)
    s2d_ref[1] = sdst.reshape(_NPAD // 16, 16)


_e2p1 = pl.pallas_call(
    _e2p1_body,
    out_shape=[
        jax.ShapeDtypeStruct((_NPAD, _D), jnp.float32),
        jax.ShapeDtypeStruct((_NPAD, _D), jnp.float32),
        jax.ShapeDtypeStruct((_NPAD, 1), jnp.float32),
        jax.ShapeDtypeStruct((_NPAD, 1), jnp.float32),
        jax.ShapeDtypeStruct((2, _NPAD // 16, 16), jnp.float32),
    ],
)


_RBW = _NPAD // _G


def _e2head_body(gat_ref, ps_ref, pq_ref, g_ref, be_ref, hprev_ref,
                 batch_ref, wq_ref, bq_ref, wr_ref, br_ref, out_ref,
                 hmax_ref, band_ref, hbuf_ref):
    h = _bn_res(gat_ref, ps_ref, pq_ref, g_ref, be_ref, hprev_ref)
    hbuf_ref[...] = h
    batch = batch_ref[...]
    gidx_row = lax.broadcasted_iota(jnp.int32, (_NPAD, _G), 1)
    oh = (batch == gidx_row).astype(jnp.float32)
    dn = (((0,), (0,)), ((), ()))
    sums = lax.dot_general(oh, h, dn, preferred_element_type=jnp.float32)
    cnt = lax.dot_general(oh, jnp.ones((_NPAD, 1), jnp.float32), dn,
                          preferred_element_type=jnp.float32)
    mean = sums / jnp.maximum(cnt, 1.0)

    lt = (batch < gidx_row).astype(jnp.float32)
    starts = lax.dot_general(jnp.ones((_NPAD, 1), jnp.float32), lt, dn,
                             preferred_element_type=jnp.float32)
    counts = cnt[:, 0].reshape(1, _G)
    giota = lax.broadcasted_iota(jnp.int32, (1, _G), 1)
    neg_inf = jnp.float32(-jnp.inf)

    def band_body(k, carry):
        band_ref[pl.ds(k, 1), :] = jnp.max(
            hbuf_ref[pl.ds(k * _RBW, _RBW), :], axis=0, keepdims=True)
        return carry

    lax.fori_loop(0, _G, band_body, 0)

    biota = lax.broadcasted_iota(jnp.int32, (_G, _D), 0)
    wiota = lax.broadcasted_iota(jnp.int32, (_RBW, _D), 0)

    def mx_body(gi, carry):
        sel = (giota == gi).astype(jnp.float32)
        st = jnp.sum(starts * sel).astype(jnp.int32)
        cn = jnp.sum(counts * sel).astype(jnp.int32)
        en = st + cn
        js = (st + _RBW - 1) // _RBW
        je = en // _RBW
        inner = jnp.max(jnp.where((biota >= js) & (biota < je),
                                  band_ref[...], neg_inf), axis=0)
        c1 = jnp.minimum(st, _NPAD - _RBW)
        w1 = hbuf_ref[pl.ds(c1, _RBW), :]
        r1 = wiota + c1
        m1 = jnp.max(jnp.where((r1 >= st) & (r1 < en), w1, neg_inf), axis=0)
        c2 = jnp.clip(en - _RBW, 0, _NPAD - _RBW)
        w2 = hbuf_ref[pl.ds(c2, _RBW), :]
        r2 = wiota + c2
        m2 = jnp.max(jnp.where((r2 >= st) & (r2 < en), w2, neg_inf), axis=0)
        hm = jnp.maximum(jnp.maximum(inner, m1), m2)
        hmax_ref[pl.ds(gi, 1), :] = hm.reshape(1, _D)
        return carry

    lax.fori_loop(0, _G, mx_body, 0)
    hmax = hmax_ref[...]
    hmax = jnp.where(jnp.isfinite(hmax), hmax, 0.0)
    pooled = jnp.concatenate([mean, hmax], axis=1)
    hq = jnp.tanh(jnp.dot(pooled, wq_ref[...],
                          preferred_element_type=jnp.float32) + bq_ref[...])
    comb = jnp.concatenate([pooled, hq], axis=1)
    out_ref[...] = jnp.dot(comb, wr_ref[...],
                           preferred_element_type=jnp.float32) + br_ref[...]


_e2head = pl.pallas_call(
    _e2head_body,
    out_shape=jax.ShapeDtypeStruct((_G, 10), jnp.float32),
    scratch_shapes=[pltpu.VMEM((_G, _D), jnp.float32),
                    pltpu.VMEM((_G, _D), jnp.float32),
                    pltpu.VMEM((_NPAD, _D), jnp.float32)],
)


def kernel(x, edge_index, batch, W1, asrc1, adst1, b1, g1, be1,
           W2, asrc2, adst2, b2, g2, be2, Wq, bq, Wr, br):
    src = edge_index[0]
    dst = edge_index[1]
    xp = jnp.pad(x, ((0, _NPAD - _N), (0, 0)))
    batch_p = jnp.pad(batch, (0, _NPAD - _N),
                      constant_values=_G).reshape(_NPAD, 1)
    av1 = jnp.stack([asrc1, adst1], axis=1)
    av2 = jnp.stack([asrc2, adst2], axis=1)

    xw, ssrc, sdst, s2d = _p1(xp, W1, av1)
    t0, t1 = _p2(xw, ssrc, ssrc)
    gi, si = _index(src, dst, s2d)
    o0, o1 = _stream(gi, si, t0, t1)
    gat, ps, pq = _e1(o0, o0, o1, o1,
                      ssrc, sdst, sdst, b1.reshape(1, _D))
    h1, xw, ssrc, sdst, s2d = _e2p1(gat, ps, pq, g1.reshape(1, _D),
                                    be1.reshape(1, _D), xp, W2, av2)
    t0, t1 = _p2(xw, ssrc, ssrc)
    gi, si = _index(src, dst, s2d)
    o0, o1 = _stream(gi, si, t0, t1)
    gat, ps, pq = _e1(o0, o0, o1, o1,
                      ssrc, sdst, sdst, b2.reshape(1, _D))
    return _e2head(gat, ps, pq, g2.reshape(1, _D), be2.reshape(1, _D), h1,
                   batch_p, Wq, bq.reshape(1, -1), Wr, br.reshape(1, -1))
